# baseline (speedup 1.0000x reference)
import functools

import jax
import jax.numpy as jnp
from jax import lax
from jax.experimental import pallas as pl
from jax.experimental.pallas import tpu as pltpu
from jax.experimental.pallas import tpu_sc as plsc

N = 10000
E = 320000
D_IN = 128
D_HID = 128
D_OUT = 237

NPAD = 10240
NC, NS = 2, 16
NW = NC * NS
CH = 128
KPW = 80
EPAD = NW * KPW * CH
RPT = NPAD // NS
HK = KPW // 2
KS = 48
KROWS = 2 * KS

_mesh = plsc.VectorSubcoreMesh(core_axis_name="c", subcore_axis_name="s")


@functools.partial(
    pl.kernel,
    mesh=_mesh,
    out_type=jax.ShapeDtypeStruct((2 * NPAD, D_IN), jnp.float32),
    scratch_types=[
        pltpu.VMEM((KS, CH), jnp.int32),
        pltpu.VMEM((KS, CH), jnp.int32),
        pltpu.VMEM((CH, D_IN), jnp.float32),
        pltpu.VMEM((CH, D_IN), jnp.float32),
        pltpu.VMEM_SHARED((NPAD, D_IN), jnp.float32),
        pltpu.SemaphoreType.DMA,
        pltpu.SemaphoreType.DMA,
        pltpu.SemaphoreType.DMA,
        pltpu.SemaphoreType.DMA,
    ],
)
def _sc_aggregate(table, src_hbm, dst_hbm, zeros_hbm, out, src_v, dst_v,
                  b0, b1, acc, gs0, gs1, gs2, gs3):
    cid = lax.axis_index("c")
    sid = lax.axis_index("s")
    wid = sid * NC + cid
    pltpu.sync_copy(zeros_hbm.at[pl.ds(sid * RPT, RPT)],
                    acc.at[pl.ds(sid * RPT, RPT)])
    plsc.subcore_barrier()

    def half(h, c):
        pltpu.sync_copy(src_hbm.at[wid, pl.ds(h * KS, KS)], src_v)
        pltpu.sync_copy(dst_hbm.at[wid, pl.ds(h * KS, KS)], dst_v)

        def body(i, c2):
            j = 2 * i
            HC = CH // 2
            g0a = pltpu.async_copy(
                table.at[src_v.at[j, pl.ds(0, HC)]], b0.at[pl.ds(0, HC)], gs0)
            g0b = pltpu.async_copy(
                table.at[src_v.at[j, pl.ds(HC, HC)]], b0.at[pl.ds(HC, HC)],
                gs1)
            g1a = pltpu.async_copy(
                table.at[src_v.at[j + 1, pl.ds(0, HC)]], b1.at[pl.ds(0, HC)],
                gs2)
            g1b = pltpu.async_copy(
                table.at[src_v.at[j + 1, pl.ds(HC, HC)]],
                b1.at[pl.ds(HC, HC)], gs3)
            g0a.wait()
            g0b.wait()
            pltpu.sync_copy(b0, acc.at[dst_v.at[j]], add=True)
            g1a.wait()
            g1b.wait()
            pltpu.sync_copy(b1, acc.at[dst_v.at[j + 1]], add=True)
            return c2

        lax.fori_loop(0, HK // 2, body, 0)
        return c

    lax.fori_loop(0, 2, half, 0)
    plsc.subcore_barrier()
    pltpu.sync_copy(acc.at[pl.ds(sid * RPT, RPT)],
                    out.at[pl.ds(cid * NPAD + sid * RPT, RPT)])


@functools.partial(
    pl.kernel,
    mesh=_mesh,
    out_type=jax.ShapeDtypeStruct((2 * NPAD, D_IN), jnp.float32),
    scratch_types=[
        pltpu.VMEM((KROWS, CH), jnp.int32),
        pltpu.VMEM((CH, D_IN), jnp.float32),
        pltpu.VMEM_SHARED((NPAD, D_IN), jnp.float32),
    ],
)
def _sc_count(dst_hbm, ones_hbm, zeros_hbm, out, dst_v, ones_v, acc):
    cid = lax.axis_index("c")
    sid = lax.axis_index("s")
    wid = sid * NC + cid
    pltpu.sync_copy(dst_hbm.at[wid], dst_v)
    pltpu.sync_copy(ones_hbm, ones_v)
    pltpu.sync_copy(zeros_hbm.at[pl.ds(sid * RPT, RPT)],
                    acc.at[pl.ds(sid * RPT, RPT)])
    plsc.subcore_barrier()

    def body(j, c):
        pltpu.sync_copy(ones_v, acc.at[dst_v.at[j]], add=True)
        return c

    lax.fori_loop(0, KROWS, body, 0)
    plsc.subcore_barrier()
    pltpu.sync_copy(acc.at[pl.ds(sid * RPT, RPT)],
                    out.at[pl.ds(cid * NPAD + sid * RPT, RPT)])


_RB = 2560


def _scale_body(c0_ref, c1_ref, x_ref, xs_ref, dis_ref):
    deg = c0_ref[...] + c1_ref[...] + 1.0
    dis = lax.rsqrt(deg)
    dis_ref[...] = dis
    xs_ref[...] = x_ref[...] * dis


def _tc_scale(c0, c1, xp):
    grid = NPAD // _RB
    return pl.pallas_call(
        _scale_body,
        grid=(grid,),
        in_specs=[
            pl.BlockSpec((_RB, 1), lambda i: (i, 0)),
            pl.BlockSpec((_RB, 1), lambda i: (i, 0)),
            pl.BlockSpec((_RB, D_IN), lambda i: (i, 0)),
        ],
        out_specs=[
            pl.BlockSpec((_RB, D_IN), lambda i: (i, 0)),
            pl.BlockSpec((_RB, 1), lambda i: (i, 0)),
        ],
        out_shape=[
            jax.ShapeDtypeStruct((NPAD, D_IN), jnp.float32),
            jax.ShapeDtypeStruct((NPAD, 1), jnp.float32),
        ],
    )(c0, c1, xp)


def _mm1_body(a0_ref, a1_ref, xs_ref, dis_ref, w_ref, b_ref, o_ref):
    dis = dis_ref[...]
    t = dis * (a0_ref[...] + a1_ref[...] + xs_ref[...])
    h = jnp.dot(t, w_ref[...], preferred_element_type=jnp.float32)
    h = jnp.maximum(h + b_ref[...], 0.0)
    o_ref[...] = h * dis


def _tc_mm1(a0, a1, xs, dis, w, b):
    grid = NPAD // _RB
    return pl.pallas_call(
        _mm1_body,
        grid=(grid,),
        in_specs=[
            pl.BlockSpec((_RB, D_IN), lambda i: (i, 0)),
            pl.BlockSpec((_RB, D_IN), lambda i: (i, 0)),
            pl.BlockSpec((_RB, D_IN), lambda i: (i, 0)),
            pl.BlockSpec((_RB, 1), lambda i: (i, 0)),
            pl.BlockSpec((D_IN, D_HID), lambda i: (0, 0)),
            pl.BlockSpec((1, D_HID), lambda i: (0, 0)),
        ],
        out_specs=pl.BlockSpec((_RB, D_HID), lambda i: (i, 0)),
        out_shape=jax.ShapeDtypeStruct((NPAD, D_HID), jnp.float32),
    )(a0, a1, xs, dis, w, b)


_DOP = 256


def _mm2_body(a0_ref, a1_ref, hs_ref, dis_ref, w_ref, b_ref, o_ref):
    t = dis_ref[...] * (a0_ref[...] + a1_ref[...] + hs_ref[...])
    h = jnp.dot(t, w_ref[...], preferred_element_type=jnp.float32)
    o_ref[...] = h + b_ref[...]


def _tc_mm2(a0, a1, hs, dis, w, b):
    grid = NPAD // _RB
    return pl.pallas_call(
        _mm2_body,
        grid=(grid,),
        in_specs=[
            pl.BlockSpec((_RB, D_HID), lambda i: (i, 0)),
            pl.BlockSpec((_RB, D_HID), lambda i: (i, 0)),
            pl.BlockSpec((_RB, D_HID), lambda i: (i, 0)),
            pl.BlockSpec((_RB, 1), lambda i: (i, 0)),
            pl.BlockSpec((D_HID, _DOP), lambda i: (0, 0)),
            pl.BlockSpec((1, _DOP), lambda i: (0, 0)),
        ],
        out_specs=pl.BlockSpec((_RB, _DOP), lambda i: (i, 0)),
        out_shape=jax.ShapeDtypeStruct((NPAD, _DOP), jnp.float32),
    )(a0, a1, hs, dis, w, b)


def kernel(x, edge_index, W1, b1, W2, b2):
    def stage(idx):
        idx = idx.astype(jnp.int32)
        pad_idx = jnp.full((EPAD - E,), N, jnp.int32)
        idx = jnp.concatenate([idx, pad_idx]).reshape(NW, KPW, CH)
        tail = jnp.full((NW, KS - HK, CH), N, jnp.int32)
        return jnp.concatenate(
            [idx[:, :HK], tail, idx[:, HK:], tail], axis=1)

    src = stage(edge_index[0])
    dst = stage(edge_index[1])

    xp = jnp.pad(x, ((0, NPAD - N), (0, 0)))
    zeros = jnp.zeros((NPAD, D_IN), jnp.float32)
    ones = jnp.ones((CH, D_IN), jnp.float32)

    cnt = _sc_count(dst, ones, zeros)
    xs, dis = _tc_scale(cnt[:NPAD, :1], cnt[NPAD:, :1], xp)

    agg1 = _sc_aggregate(xs, src, dst, zeros)
    h1s = _tc_mm1(agg1[:NPAD], agg1[NPAD:], xs, dis, W1,
                  b1.reshape(1, D_HID))

    agg2 = _sc_aggregate(h1s, src, dst, zeros)
    w2p = jnp.pad(W2, ((0, 0), (0, _DOP - D_OUT)))
    b2p = jnp.pad(b2, (0, _DOP - D_OUT)).reshape(1, _DOP)
    out = _tc_mm2(agg2[:NPAD], agg2[NPAD:], h1s, dis, w2p, b2p)
    return out[:N, :D_OUT]


# speedup vs baseline: 1.0009x; 1.0009x over previous
import functools

import jax
import jax.numpy as jnp
from jax import lax
from jax.experimental import pallas as pl
from jax.experimental.pallas import tpu as pltpu
from jax.experimental.pallas import tpu_sc as plsc

N = 10000
E = 320000
D_IN = 128
D_HID = 128
D_OUT = 237

NPAD = 10240
NC, NS = 2, 16
NW = NC * NS
CH = 128
KPW = 80
EPAD = NW * KPW * CH
RPT = NPAD // NS
HK = KPW // 2
KS = 48
KROWS = 2 * KS

_mesh = plsc.VectorSubcoreMesh(core_axis_name="c", subcore_axis_name="s")


@functools.partial(
    pl.kernel,
    mesh=_mesh,
    out_type=jax.ShapeDtypeStruct((2 * NPAD, D_IN), jnp.float32),
    scratch_types=[
        pltpu.VMEM((KS, CH), jnp.int32),
        pltpu.VMEM((KS, CH), jnp.int32),
        pltpu.VMEM((CH, D_IN), jnp.float32),
        pltpu.VMEM((CH, D_IN), jnp.float32),
        pltpu.VMEM_SHARED((NPAD, D_IN), jnp.float32),
        pltpu.SemaphoreType.DMA,
        pltpu.SemaphoreType.DMA,
    ],
)
def _sc_aggregate(table, src_hbm, dst_hbm, zeros_hbm, out, src_v, dst_v,
                  b0, b1, acc, gs0, gs1):
    cid = lax.axis_index("c")
    sid = lax.axis_index("s")
    wid = sid * NC + cid
    pltpu.sync_copy(zeros_hbm.at[pl.ds(sid * RPT, RPT)],
                    acc.at[pl.ds(sid * RPT, RPT)])
    plsc.subcore_barrier()

    def half(h, c):
        pltpu.sync_copy(src_hbm.at[wid, pl.ds(h * KS, KS)], src_v)
        pltpu.sync_copy(dst_hbm.at[wid, pl.ds(h * KS, KS)], dst_v)

        def body(i, c2):
            j = 2 * i
            g0 = pltpu.async_copy(table.at[src_v.at[j]], b0, gs0)
            g1 = pltpu.async_copy(table.at[src_v.at[j + 1]], b1, gs1)
            g0.wait()
            pltpu.sync_copy(b0, acc.at[dst_v.at[j]], add=True)
            g1.wait()
            pltpu.sync_copy(b1, acc.at[dst_v.at[j + 1]], add=True)
            return c2

        lax.fori_loop(0, HK // 2, body, 0)
        return c

    lax.fori_loop(0, 2, half, 0)
    plsc.subcore_barrier()
    pltpu.sync_copy(acc.at[pl.ds(sid * RPT, RPT)],
                    out.at[pl.ds(cid * NPAD + sid * RPT, RPT)])


@functools.partial(
    pl.kernel,
    mesh=_mesh,
    out_type=jax.ShapeDtypeStruct((2 * NPAD, D_IN), jnp.float32),
    scratch_types=[
        pltpu.VMEM((KROWS, CH), jnp.int32),
        pltpu.VMEM((CH, D_IN), jnp.float32),
        pltpu.VMEM_SHARED((NPAD, D_IN), jnp.float32),
    ],
)
def _sc_count(dst_hbm, ones_hbm, zeros_hbm, out, dst_v, ones_v, acc):
    cid = lax.axis_index("c")
    sid = lax.axis_index("s")
    wid = sid * NC + cid
    pltpu.sync_copy(dst_hbm.at[wid], dst_v)
    pltpu.sync_copy(ones_hbm, ones_v)
    pltpu.sync_copy(zeros_hbm.at[pl.ds(sid * RPT, RPT)],
                    acc.at[pl.ds(sid * RPT, RPT)])
    plsc.subcore_barrier()

    def body(j, c):
        pltpu.sync_copy(ones_v, acc.at[dst_v.at[j]], add=True)
        return c

    lax.fori_loop(0, KROWS, body, 0)
    plsc.subcore_barrier()
    pltpu.sync_copy(acc.at[pl.ds(sid * RPT, RPT)],
                    out.at[pl.ds(cid * NPAD + sid * RPT, RPT)])


_RB = 2560


def _scale_body(c0_ref, c1_ref, x_ref, xs_ref, dis_ref):
    deg = c0_ref[...] + c1_ref[...] + 1.0
    dis = lax.rsqrt(deg)
    dis_ref[...] = dis
    xs_ref[...] = x_ref[...] * dis


def _tc_scale(c0, c1, xp):
    grid = NPAD // _RB
    return pl.pallas_call(
        _scale_body,
        grid=(grid,),
        in_specs=[
            pl.BlockSpec((_RB, 1), lambda i: (i, 0)),
            pl.BlockSpec((_RB, 1), lambda i: (i, 0)),
            pl.BlockSpec((_RB, D_IN), lambda i: (i, 0)),
        ],
        out_specs=[
            pl.BlockSpec((_RB, D_IN), lambda i: (i, 0)),
            pl.BlockSpec((_RB, 1), lambda i: (i, 0)),
        ],
        out_shape=[
            jax.ShapeDtypeStruct((NPAD, D_IN), jnp.float32),
            jax.ShapeDtypeStruct((NPAD, 1), jnp.float32),
        ],
    )(c0, c1, xp)


def _mm1_body(a0_ref, a1_ref, xs_ref, dis_ref, w_ref, b_ref, o_ref):
    dis = dis_ref[...]
    t = dis * (a0_ref[...] + a1_ref[...] + xs_ref[...])
    h = jnp.dot(t, w_ref[...], preferred_element_type=jnp.float32)
    h = jnp.maximum(h + b_ref[...], 0.0)
    o_ref[...] = h * dis


def _tc_mm1(a0, a1, xs, dis, w, b):
    grid = NPAD // _RB
    return pl.pallas_call(
        _mm1_body,
        grid=(grid,),
        in_specs=[
            pl.BlockSpec((_RB, D_IN), lambda i: (i, 0)),
            pl.BlockSpec((_RB, D_IN), lambda i: (i, 0)),
            pl.BlockSpec((_RB, D_IN), lambda i: (i, 0)),
            pl.BlockSpec((_RB, 1), lambda i: (i, 0)),
            pl.BlockSpec((D_IN, D_HID), lambda i: (0, 0)),
            pl.BlockSpec((1, D_HID), lambda i: (0, 0)),
        ],
        out_specs=pl.BlockSpec((_RB, D_HID), lambda i: (i, 0)),
        out_shape=jax.ShapeDtypeStruct((NPAD, D_HID), jnp.float32),
    )(a0, a1, xs, dis, w, b)


_DOP = 256


def _mm2_body(a0_ref, a1_ref, hs_ref, dis_ref, w_ref, b_ref, o_ref):
    t = dis_ref[...] * (a0_ref[...] + a1_ref[...] + hs_ref[...])
    h = jnp.dot(t, w_ref[...], preferred_element_type=jnp.float32)
    o_ref[...] = h + b_ref[...]


def _tc_mm2(a0, a1, hs, dis, w, b):
    grid = NPAD // _RB
    return pl.pallas_call(
        _mm2_body,
        grid=(grid,),
        in_specs=[
            pl.BlockSpec((_RB, D_HID), lambda i: (i, 0)),
            pl.BlockSpec((_RB, D_HID), lambda i: (i, 0)),
            pl.BlockSpec((_RB, D_HID), lambda i: (i, 0)),
            pl.BlockSpec((_RB, 1), lambda i: (i, 0)),
            pl.BlockSpec((D_HID, _DOP), lambda i: (0, 0)),
            pl.BlockSpec((1, _DOP), lambda i: (0, 0)),
        ],
        out_specs=pl.BlockSpec((_RB, _DOP), lambda i: (i, 0)),
        out_shape=jax.ShapeDtypeStruct((NPAD, _DOP), jnp.float32),
    )(a0, a1, hs, dis, w, b)


def kernel(x, edge_index, W1, b1, W2, b2):
    def stage(idx):
        idx = idx.astype(jnp.int32)
        pad_idx = jnp.full((EPAD - E,), N, jnp.int32)
        idx = jnp.concatenate([idx, pad_idx]).reshape(NW, KPW, CH)
        tail = jnp.full((NW, KS - HK, CH), N, jnp.int32)
        return jnp.concatenate(
            [idx[:, :HK], tail, idx[:, HK:], tail], axis=1)

    src = stage(edge_index[0])
    dst = stage(edge_index[1])

    xp = jnp.pad(x, ((0, NPAD - N), (0, 0)))
    zeros = jnp.zeros((NPAD, D_IN), jnp.float32)
    ones = jnp.ones((CH, D_IN), jnp.float32)

    cnt = _sc_count(dst, ones, zeros)
    xs, dis = _tc_scale(cnt[:NPAD, :1], cnt[NPAD:, :1], xp)

    agg1 = _sc_aggregate(xs, src, dst, zeros)
    h1s = _tc_mm1(agg1[:NPAD], agg1[NPAD:], xs, dis, W1,
                  b1.reshape(1, D_HID))

    agg2 = _sc_aggregate(h1s, src, dst, zeros)
    w2p = jnp.pad(W2, ((0, 0), (0, _DOP - D_OUT)))
    b2p = jnp.pad(b2, (0, _DOP - D_OUT)).reshape(1, _DOP)
    out = _tc_mm2(agg2[:NPAD], agg2[NPAD:], h1s, dis, w2p, b2p)
    return out[:N, :D_OUT]


# speedup vs baseline: 1.0395x; 1.0386x over previous
import functools

import jax
import jax.numpy as jnp
from jax import lax
from jax.experimental import pallas as pl
from jax.experimental.pallas import tpu as pltpu
from jax.experimental.pallas import tpu_sc as plsc

N = 10000
E = 320000
D_IN = 128
D_HID = 128
D_OUT = 237

NPAD = 10240
NC, NS = 2, 16
NW = NC * NS
CH = 128
KPW = 80
EPAD = NW * KPW * CH
RPT = NPAD // NS
HK = KPW // 2
KS = 48
KROWS = 2 * KS

_mesh = plsc.VectorSubcoreMesh(core_axis_name="c", subcore_axis_name="s")


@functools.partial(
    pl.kernel,
    mesh=_mesh,
    out_type=jax.ShapeDtypeStruct((2 * NPAD, D_IN), jnp.float32),
    scratch_types=[
        pltpu.VMEM((KS, CH), jnp.int32),
        pltpu.VMEM((KS, CH), jnp.int32),
        pltpu.VMEM((CH, D_IN), jnp.float32),
        pltpu.VMEM((CH, D_IN), jnp.float32),
        pltpu.VMEM_SHARED((NPAD, D_IN), jnp.float32),
        pltpu.SemaphoreType.DMA,
        pltpu.SemaphoreType.DMA,
    ],
)
def _sc_aggregate(table, src_hbm, dst_hbm, zeros_hbm, out, src_v, dst_v,
                  b0, b1, acc, gs0, gs1):
    cid = lax.axis_index("c")
    sid = lax.axis_index("s")
    wid = sid * NC + cid
    pltpu.sync_copy(zeros_hbm.at[pl.ds(sid * RPT, RPT)],
                    acc.at[pl.ds(sid * RPT, RPT)])
    plsc.subcore_barrier()

    def half(h, c):
        pltpu.sync_copy(src_hbm.at[wid, pl.ds(h * KS, KS)], src_v)
        pltpu.sync_copy(dst_hbm.at[wid, pl.ds(h * KS, KS)], dst_v)

        def body(i, c2):
            j = 4 * i
            g0 = pltpu.async_copy(table.at[src_v.at[j]], b0, gs0)
            g1 = pltpu.async_copy(table.at[src_v.at[j + 1]], b1, gs1)
            g0.wait()
            pltpu.sync_copy(b0, acc.at[dst_v.at[j]], add=True)
            g2 = pltpu.async_copy(table.at[src_v.at[j + 2]], b0, gs0)
            g1.wait()
            pltpu.sync_copy(b1, acc.at[dst_v.at[j + 1]], add=True)
            g3 = pltpu.async_copy(table.at[src_v.at[j + 3]], b1, gs1)
            g2.wait()
            pltpu.sync_copy(b0, acc.at[dst_v.at[j + 2]], add=True)
            g3.wait()
            pltpu.sync_copy(b1, acc.at[dst_v.at[j + 3]], add=True)
            return c2

        lax.fori_loop(0, HK // 4, body, 0)
        return c

    lax.fori_loop(0, 2, half, 0)
    plsc.subcore_barrier()
    pltpu.sync_copy(acc.at[pl.ds(sid * RPT, RPT)],
                    out.at[pl.ds(cid * NPAD + sid * RPT, RPT)])


@functools.partial(
    pl.kernel,
    mesh=_mesh,
    out_type=jax.ShapeDtypeStruct((2 * NPAD, D_IN), jnp.float32),
    scratch_types=[
        pltpu.VMEM((KROWS, CH), jnp.int32),
        pltpu.VMEM((CH, D_IN), jnp.float32),
        pltpu.VMEM_SHARED((NPAD, D_IN), jnp.float32),
    ],
)
def _sc_count(dst_hbm, ones_hbm, zeros_hbm, out, dst_v, ones_v, acc):
    cid = lax.axis_index("c")
    sid = lax.axis_index("s")
    wid = sid * NC + cid
    pltpu.sync_copy(dst_hbm.at[wid], dst_v)
    pltpu.sync_copy(ones_hbm, ones_v)
    pltpu.sync_copy(zeros_hbm.at[pl.ds(sid * RPT, RPT)],
                    acc.at[pl.ds(sid * RPT, RPT)])
    plsc.subcore_barrier()

    def body(j, c):
        pltpu.sync_copy(ones_v, acc.at[dst_v.at[j]], add=True)
        return c

    lax.fori_loop(0, KROWS, body, 0)
    plsc.subcore_barrier()
    pltpu.sync_copy(acc.at[pl.ds(sid * RPT, RPT)],
                    out.at[pl.ds(cid * NPAD + sid * RPT, RPT)])


_RB = 2560


def _scale_body(c0_ref, c1_ref, x_ref, xs_ref, dis_ref):
    deg = c0_ref[...] + c1_ref[...] + 1.0
    dis = lax.rsqrt(deg)
    dis_ref[...] = dis
    xs_ref[...] = x_ref[...] * dis


def _tc_scale(c0, c1, xp):
    grid = NPAD // _RB
    return pl.pallas_call(
        _scale_body,
        grid=(grid,),
        in_specs=[
            pl.BlockSpec((_RB, 1), lambda i: (i, 0)),
            pl.BlockSpec((_RB, 1), lambda i: (i, 0)),
            pl.BlockSpec((_RB, D_IN), lambda i: (i, 0)),
        ],
        out_specs=[
            pl.BlockSpec((_RB, D_IN), lambda i: (i, 0)),
            pl.BlockSpec((_RB, 1), lambda i: (i, 0)),
        ],
        out_shape=[
            jax.ShapeDtypeStruct((NPAD, D_IN), jnp.float32),
            jax.ShapeDtypeStruct((NPAD, 1), jnp.float32),
        ],
    )(c0, c1, xp)


def _mm1_body(a0_ref, a1_ref, xs_ref, dis_ref, w_ref, b_ref, o_ref):
    dis = dis_ref[...]
    t = dis * (a0_ref[...] + a1_ref[...] + xs_ref[...])
    h = jnp.dot(t, w_ref[...], preferred_element_type=jnp.float32)
    h = jnp.maximum(h + b_ref[...], 0.0)
    o_ref[...] = h * dis


def _tc_mm1(a0, a1, xs, dis, w, b):
    grid = NPAD // _RB
    return pl.pallas_call(
        _mm1_body,
        grid=(grid,),
        in_specs=[
            pl.BlockSpec((_RB, D_IN), lambda i: (i, 0)),
            pl.BlockSpec((_RB, D_IN), lambda i: (i, 0)),
            pl.BlockSpec((_RB, D_IN), lambda i: (i, 0)),
            pl.BlockSpec((_RB, 1), lambda i: (i, 0)),
            pl.BlockSpec((D_IN, D_HID), lambda i: (0, 0)),
            pl.BlockSpec((1, D_HID), lambda i: (0, 0)),
        ],
        out_specs=pl.BlockSpec((_RB, D_HID), lambda i: (i, 0)),
        out_shape=jax.ShapeDtypeStruct((NPAD, D_HID), jnp.float32),
    )(a0, a1, xs, dis, w, b)


_DOP = 256


def _mm2_body(a0_ref, a1_ref, hs_ref, dis_ref, w_ref, b_ref, o_ref):
    t = dis_ref[...] * (a0_ref[...] + a1_ref[...] + hs_ref[...])
    h = jnp.dot(t, w_ref[...], preferred_element_type=jnp.float32)
    o_ref[...] = h + b_ref[...]


def _tc_mm2(a0, a1, hs, dis, w, b):
    grid = NPAD // _RB
    return pl.pallas_call(
        _mm2_body,
        grid=(grid,),
        in_specs=[
            pl.BlockSpec((_RB, D_HID), lambda i: (i, 0)),
            pl.BlockSpec((_RB, D_HID), lambda i: (i, 0)),
            pl.BlockSpec((_RB, D_HID), lambda i: (i, 0)),
            pl.BlockSpec((_RB, 1), lambda i: (i, 0)),
            pl.BlockSpec((D_HID, _DOP), lambda i: (0, 0)),
            pl.BlockSpec((1, _DOP), lambda i: (0, 0)),
        ],
        out_specs=pl.BlockSpec((_RB, _DOP), lambda i: (i, 0)),
        out_shape=jax.ShapeDtypeStruct((NPAD, _DOP), jnp.float32),
    )(a0, a1, hs, dis, w, b)


def kernel(x, edge_index, W1, b1, W2, b2):
    def stage(idx):
        idx = idx.astype(jnp.int32)
        pad_idx = jnp.full((EPAD - E,), N, jnp.int32)
        idx = jnp.concatenate([idx, pad_idx]).reshape(NW, KPW, CH)
        tail = jnp.full((NW, KS - HK, CH), N, jnp.int32)
        return jnp.concatenate(
            [idx[:, :HK], tail, idx[:, HK:], tail], axis=1)

    src = stage(edge_index[0])
    dst = stage(edge_index[1])

    xp = jnp.pad(x, ((0, NPAD - N), (0, 0)))
    zeros = jnp.zeros((NPAD, D_IN), jnp.float32)
    ones = jnp.ones((CH, D_IN), jnp.float32)

    cnt = _sc_count(dst, ones, zeros)
    xs, dis = _tc_scale(cnt[:NPAD, :1], cnt[NPAD:, :1], xp)

    agg1 = _sc_aggregate(xs, src, dst, zeros)
    h1s = _tc_mm1(agg1[:NPAD], agg1[NPAD:], xs, dis, W1,
                  b1.reshape(1, D_HID))

    agg2 = _sc_aggregate(h1s, src, dst, zeros)
    w2p = jnp.pad(W2, ((0, 0), (0, _DOP - D_OUT)))
    b2p = jnp.pad(b2, (0, _DOP - D_OUT)).reshape(1, _DOP)
    out = _tc_mm2(agg2[:NPAD], agg2[NPAD:], h1s, dis, w2p, b2p)
    return out[:N, :D_OUT]


# speedup vs baseline: 1.0697x; 1.0290x over previous
import functools

import jax
import jax.numpy as jnp
from jax import lax
from jax.experimental import pallas as pl
from jax.experimental.pallas import tpu as pltpu
from jax.experimental.pallas import tpu_sc as plsc

N = 10000
E = 320000
D_IN = 128
D_HID = 128
D_OUT = 237

NPAD = 10240
NC, NS = 2, 16
NW = NC * NS
CH = 128
KPW = 80
EPAD = NW * KPW * CH
RPT = NPAD // NS
HK = KPW // 2
KS = 48
KROWS = 2 * KS

_mesh = plsc.VectorSubcoreMesh(core_axis_name="c", subcore_axis_name="s")


@functools.partial(
    pl.kernel,
    mesh=_mesh,
    out_type=jax.ShapeDtypeStruct((2 * NPAD, D_IN), jnp.float32),
    scratch_types=[
        pltpu.VMEM((KS, CH), jnp.int32),
        pltpu.VMEM((KS, CH), jnp.int32),
        pltpu.VMEM((CH, D_IN), jnp.float32),
        pltpu.VMEM((CH, D_IN), jnp.float32),
        pltpu.VMEM_SHARED((NPAD, D_IN), jnp.float32),
        pltpu.SemaphoreType.DMA,
        pltpu.SemaphoreType.DMA,
    ],
)
def _sc_aggregate(table, src_hbm, dst_hbm, zeros_hbm, out, src_v, dst_v,
                  b0, b1, acc, gs0, gs1):
    cid = lax.axis_index("c")
    sid = lax.axis_index("s")
    wid = sid * NC + cid
    pltpu.sync_copy(zeros_hbm.at[pl.ds(sid * RPT, RPT)],
                    acc.at[pl.ds(sid * RPT, RPT)])
    plsc.subcore_barrier()

    def half(h, c):
        pltpu.sync_copy(src_hbm.at[wid, pl.ds(h * KS, KS)], src_v)
        pltpu.sync_copy(dst_hbm.at[wid, pl.ds(h * KS, KS)], dst_v)

        UNROLL = 8

        def body(i, c2):
            j = UNROLL * i
            bufs = (b0, b1)
            sems = (gs0, gs1)
            pend = [None, None]
            for k in range(UNROLL):
                s = k % 2
                if pend[s] is not None:
                    pend[s].wait()
                    pltpu.sync_copy(bufs[s], acc.at[dst_v.at[j + k - 2]],
                                    add=True)
                pend[s] = pltpu.async_copy(table.at[src_v.at[j + k]],
                                           bufs[s], sems[s])
            for k in range(UNROLL - 2, UNROLL):
                s = k % 2
                pend[s].wait()
                pltpu.sync_copy(bufs[s], acc.at[dst_v.at[j + k]], add=True)
            return c2

        lax.fori_loop(0, HK // UNROLL, body, 0)
        return c

    lax.fori_loop(0, 2, half, 0)
    plsc.subcore_barrier()
    pltpu.sync_copy(acc.at[pl.ds(sid * RPT, RPT)],
                    out.at[pl.ds(cid * NPAD + sid * RPT, RPT)])


@functools.partial(
    pl.kernel,
    mesh=_mesh,
    out_type=jax.ShapeDtypeStruct((2 * NPAD, D_IN), jnp.float32),
    scratch_types=[
        pltpu.VMEM((KROWS, CH), jnp.int32),
        pltpu.VMEM((CH, D_IN), jnp.float32),
        pltpu.VMEM_SHARED((NPAD, D_IN), jnp.float32),
    ],
)
def _sc_count(dst_hbm, ones_hbm, zeros_hbm, out, dst_v, ones_v, acc):
    cid = lax.axis_index("c")
    sid = lax.axis_index("s")
    wid = sid * NC + cid
    pltpu.sync_copy(dst_hbm.at[wid], dst_v)
    pltpu.sync_copy(ones_hbm, ones_v)
    pltpu.sync_copy(zeros_hbm.at[pl.ds(sid * RPT, RPT)],
                    acc.at[pl.ds(sid * RPT, RPT)])
    plsc.subcore_barrier()

    def body(j, c):
        pltpu.sync_copy(ones_v, acc.at[dst_v.at[j]], add=True)
        return c

    lax.fori_loop(0, KROWS, body, 0)
    plsc.subcore_barrier()
    pltpu.sync_copy(acc.at[pl.ds(sid * RPT, RPT)],
                    out.at[pl.ds(cid * NPAD + sid * RPT, RPT)])


_RB = 2560


def _scale_body(c0_ref, c1_ref, x_ref, xs_ref, dis_ref):
    deg = c0_ref[...] + c1_ref[...] + 1.0
    dis = lax.rsqrt(deg)
    dis_ref[...] = dis
    xs_ref[...] = x_ref[...] * dis


def _tc_scale(c0, c1, xp):
    grid = NPAD // _RB
    return pl.pallas_call(
        _scale_body,
        grid=(grid,),
        in_specs=[
            pl.BlockSpec((_RB, 1), lambda i: (i, 0)),
            pl.BlockSpec((_RB, 1), lambda i: (i, 0)),
            pl.BlockSpec((_RB, D_IN), lambda i: (i, 0)),
        ],
        out_specs=[
            pl.BlockSpec((_RB, D_IN), lambda i: (i, 0)),
            pl.BlockSpec((_RB, 1), lambda i: (i, 0)),
        ],
        out_shape=[
            jax.ShapeDtypeStruct((NPAD, D_IN), jnp.float32),
            jax.ShapeDtypeStruct((NPAD, 1), jnp.float32),
        ],
    )(c0, c1, xp)


def _mm1_body(a0_ref, a1_ref, xs_ref, dis_ref, w_ref, b_ref, o_ref):
    dis = dis_ref[...]
    t = dis * (a0_ref[...] + a1_ref[...] + xs_ref[...])
    h = jnp.dot(t, w_ref[...], preferred_element_type=jnp.float32)
    h = jnp.maximum(h + b_ref[...], 0.0)
    o_ref[...] = h * dis


def _tc_mm1(a0, a1, xs, dis, w, b):
    grid = NPAD // _RB
    return pl.pallas_call(
        _mm1_body,
        grid=(grid,),
        in_specs=[
            pl.BlockSpec((_RB, D_IN), lambda i: (i, 0)),
            pl.BlockSpec((_RB, D_IN), lambda i: (i, 0)),
            pl.BlockSpec((_RB, D_IN), lambda i: (i, 0)),
            pl.BlockSpec((_RB, 1), lambda i: (i, 0)),
            pl.BlockSpec((D_IN, D_HID), lambda i: (0, 0)),
            pl.BlockSpec((1, D_HID), lambda i: (0, 0)),
        ],
        out_specs=pl.BlockSpec((_RB, D_HID), lambda i: (i, 0)),
        out_shape=jax.ShapeDtypeStruct((NPAD, D_HID), jnp.float32),
    )(a0, a1, xs, dis, w, b)


_DOP = 256


def _mm2_body(a0_ref, a1_ref, hs_ref, dis_ref, w_ref, b_ref, o_ref):
    t = dis_ref[...] * (a0_ref[...] + a1_ref[...] + hs_ref[...])
    h = jnp.dot(t, w_ref[...], preferred_element_type=jnp.float32)
    o_ref[...] = h + b_ref[...]


def _tc_mm2(a0, a1, hs, dis, w, b):
    grid = NPAD // _RB
    return pl.pallas_call(
        _mm2_body,
        grid=(grid,),
        in_specs=[
            pl.BlockSpec((_RB, D_HID), lambda i: (i, 0)),
            pl.BlockSpec((_RB, D_HID), lambda i: (i, 0)),
            pl.BlockSpec((_RB, D_HID), lambda i: (i, 0)),
            pl.BlockSpec((_RB, 1), lambda i: (i, 0)),
            pl.BlockSpec((D_HID, _DOP), lambda i: (0, 0)),
            pl.BlockSpec((1, _DOP), lambda i: (0, 0)),
        ],
        out_specs=pl.BlockSpec((_RB, _DOP), lambda i: (i, 0)),
        out_shape=jax.ShapeDtypeStruct((NPAD, _DOP), jnp.float32),
    )(a0, a1, hs, dis, w, b)


def kernel(x, edge_index, W1, b1, W2, b2):
    def stage(idx):
        idx = idx.astype(jnp.int32)
        pad_idx = jnp.full((EPAD - E,), N, jnp.int32)
        idx = jnp.concatenate([idx, pad_idx]).reshape(NW, KPW, CH)
        tail = jnp.full((NW, KS - HK, CH), N, jnp.int32)
        return jnp.concatenate(
            [idx[:, :HK], tail, idx[:, HK:], tail], axis=1)

    src = stage(edge_index[0])
    dst = stage(edge_index[1])

    xp = jnp.pad(x, ((0, NPAD - N), (0, 0)))
    zeros = jnp.zeros((NPAD, D_IN), jnp.float32)
    ones = jnp.ones((CH, D_IN), jnp.float32)

    cnt = _sc_count(dst, ones, zeros)
    xs, dis = _tc_scale(cnt[:NPAD, :1], cnt[NPAD:, :1], xp)

    agg1 = _sc_aggregate(xs, src, dst, zeros)
    h1s = _tc_mm1(agg1[:NPAD], agg1[NPAD:], xs, dis, W1,
                  b1.reshape(1, D_HID))

    agg2 = _sc_aggregate(h1s, src, dst, zeros)
    w2p = jnp.pad(W2, ((0, 0), (0, _DOP - D_OUT)))
    b2p = jnp.pad(b2, (0, _DOP - D_OUT)).reshape(1, _DOP)
    out = _tc_mm2(agg2[:NPAD], agg2[NPAD:], h1s, dis, w2p, b2p)
    return out[:N, :D_OUT]
